# baseline (speedup 1.0000x reference)
import functools
import math

import jax
import jax.numpy as jnp
from jax import lax
from jax.experimental import pallas as pl
from jax.experimental.pallas import tpu as pltpu
from jax.experimental.pallas import tpu_sc as plsc

N = 10000
E = 320000
D = 128
NC = 4
ECH = E // NC
EB = 2560
SCB = 128
CBLK = ECH // SCB
NCORE = 2
NSUB = 16
NWORK = NCORE * NSUB
NPAD = 10240
ROWS_PER_TEC = NPAD // NSUB
INV_SQRT_AVG = 1.0 / math.sqrt(32.0)


def _silu(x):
    return x * jax.nn.sigmoid(x)


def _node_prep_body(nf_ref, sp_ref, wup_ref, wskip_ref, h_ref, sc_ref):
    nf = nf_ref[:]
    h_ref[:] = jnp.dot(nf, wup_ref[:], preferred_element_type=jnp.float32)
    sp = sp_ref[:]
    acc = jnp.zeros_like(nf)
    for k in range(5):
        masked = jnp.where(sp == k, nf, 0.0)
        acc = acc + jnp.dot(masked, wskip_ref[k], preferred_element_type=jnp.float32)
    sc_ref[:] = acc


def _node_prep(node_feats, node_specie2d, W_up, W_skip):
    return pl.pallas_call(
        _node_prep_body,
        out_shape=[
            jax.ShapeDtypeStruct((N, D), jnp.float32),
            jax.ShapeDtypeStruct((N, D), jnp.float32),
        ],
    )(node_feats, node_specie2d, W_up, W_skip)


ECHP = 81920
BPW = ECHP // (NWORK * SCB)
NBUF = 4


@functools.cache
def _make_sc_gather():
    mesh = plsc.VectorSubcoreMesh(core_axis_name="c", subcore_axis_name="s",
                                  num_cores=NCORE, num_subcores=NSUB)
    return functools.partial(
        pl.kernel,
        mesh=mesh,
        out_type=jax.ShapeDtypeStruct((ECHP, D), jnp.float32),
        scratch_types=[
            pltpu.VMEM((BPW * SCB,), jnp.int32),
            [pltpu.VMEM((SCB, D), jnp.float32) for _ in range(NBUF)],
            [pltpu.SemaphoreType.DMA for _ in range(NBUF)],
            [pltpu.SemaphoreType.DMA for _ in range(NBUF)],
        ],
    )(_sc_gather_body)


def _sc_gather_body(h_hbm, snd_hbm, out_hbm, sidx, rows, gsem, osem):
    c = lax.axis_index("c")
    t = lax.axis_index("s")
    w = t * NCORE + c
    base = w * (BPW * SCB)
    pltpu.sync_copy(snd_hbm.at[pl.ds(base, BPW * SCB)], sidx)

    def _gather(k):
        return pltpu.async_copy(
            h_hbm.at[sidx.at[pl.ds(k * SCB, SCB)]], rows[k % NBUF],
            gsem[k % NBUF])

    gh = [None] * BPW
    oh = [None] * BPW
    for k in range(NBUF):
        gh[k] = _gather(k)
    for k in range(BPW):
        p = k % NBUF
        gh[k].wait()
        oh[k] = pltpu.async_copy(
            rows[p], out_hbm.at[pl.ds(base + k * SCB, SCB)], osem[p])
        if k + NBUF < BPW:
            oh[k].wait()
            gh[k + NBUF] = _gather(k + NBUF)
    for k in range(BPW - NBUF, BPW):
        oh[k].wait()


_SH_COEF = [
    math.sqrt(3.0), math.sqrt(3.0), math.sqrt(3.0),
    math.sqrt(15.0), math.sqrt(15.0), math.sqrt(5.0) / 2.0,
    math.sqrt(15.0), math.sqrt(15.0) / 2.0,
    math.sqrt(35.0 / 8.0), math.sqrt(105.0), math.sqrt(21.0 / 8.0),
    math.sqrt(7.0) / 2.0, math.sqrt(21.0 / 8.0), math.sqrt(105.0) / 2.0,
    math.sqrt(35.0 / 8.0),
]


def _edge_tc_body(vt_ref, msgs_ref, wsh_ref, q1_ref, w2_ref, w3_ref, wd_ref,
                  y_ref):
    x = vt_ref[0:1]
    y = vt_ref[1:2]
    z = vt_ref[2:3]
    x2 = x * x + y * y + z * z
    l = jnp.sqrt(jnp.where(x2 == 0.0, 1.0, x2))
    invl = 1.0 / l
    ns = (lax.broadcasted_iota(jnp.int32, (8, 1), 0) + 1).astype(jnp.float32)
    l2 = l * l
    l5 = l2 * l2 * l
    env = 1.0 - 21.0 * l5 + 35.0 * l5 * l - 15.0 * l5 * l2
    cutoff = jnp.where(l < 1.0, env, 0.0)
    radT = (jnp.sqrt(2.0) * jnp.sin(ns * (jnp.pi * l))) * (cutoff * invl)
    ux, uy, uz = x * invl, y * invl, z * invl
    comps = [
        ux, uy, uz,
        ux * uy, uy * uz, 3.0 * uz * uz - 1.0, ux * uz, ux * ux - uy * uy,
        uy * (3.0 * ux * ux - uy * uy), ux * uy * uz,
        uy * (5.0 * uz * uz - 1.0), uz * (5.0 * uz * uz - 3.0),
        ux * (5.0 * uz * uz - 1.0), uz * (ux * ux - uy * uy),
        ux * (ux * ux - 3.0 * uy * uy),
    ]
    s = jnp.zeros_like(x)
    for k in range(15):
        s = s + (wsh_ref[0:1, k:k + 1] * _SH_COEF[k]) * comps[k]
    p = jnp.concatenate([radT, s], axis=0)
    r = lax.dot_general(p, q1_ref[:], (((0,), (0,)), ((), ())),
                        preferred_element_type=jnp.float32)
    m = _silu(r[:, :64])
    scol = r[:, 64:65]
    m = _silu(jnp.dot(m, w2_ref[:], preferred_element_type=jnp.float32))
    m = jnp.dot(m, w3_ref[:], preferred_element_type=jnp.float32)
    m = m * INV_SQRT_AVG
    msgs = msgs_ref[:]
    wd = wd_ref[:]
    y_ref[:] = (
        jnp.dot(msgs * m[:, :D], wd[:D], preferred_element_type=jnp.float32)
        + jnp.dot(msgs * (m[:, D:] * scol), wd[D:],
                  preferred_element_type=jnp.float32))


def _edge_tc(vectors_t, msgs, w_sh2d, Q1, W2, W3, W_down):
    grid = (ECHP // EB,)
    return pl.pallas_call(
        _edge_tc_body,
        grid=grid,
        in_specs=[
            pl.BlockSpec((3, EB), lambda i: (0, i)),
            pl.BlockSpec((EB, D), lambda i: (i, 0)),
            pl.BlockSpec((1, 15), lambda i: (0, 0)),
            pl.BlockSpec((9, 128), lambda i: (0, 0)),
            pl.BlockSpec((64, 64), lambda i: (0, 0)),
            pl.BlockSpec((64, 2 * D), lambda i: (0, 0)),
            pl.BlockSpec((2 * D, D), lambda i: (0, 0)),
        ],
        out_specs=pl.BlockSpec((EB, D), lambda i: (i, 0)),
        out_shape=jax.ShapeDtypeStruct((ECHP, D), jnp.float32),
    )(vectors_t, msgs, w_sh2d, Q1, W2, W3, W_down)


@functools.cache
def _make_sc_scatter():
    mesh = plsc.VectorSubcoreMesh(core_axis_name="c", subcore_axis_name="s",
                                  num_cores=NCORE, num_subcores=NSUB)
    return functools.partial(
        pl.kernel,
        mesh=mesh,
        out_type=jax.ShapeDtypeStruct((NCORE, NPAD, D), jnp.float32),
        scratch_types=[
            pltpu.VMEM((SCB,), jnp.int32),
            pltpu.VMEM((SCB, D), jnp.float32),
            pltpu.VMEM_SHARED((NPAD, D), jnp.float32),
        ],
    )(_sc_scatter_body)


def _sc_scatter_body(ya_hbm, yb_hbm, rcva_hbm, rcvb_hbm, out_hbm,
                     ridx, rows, aggsh):
    c = lax.axis_index("c")
    t = lax.axis_index("s")

    def _zero_row(r, carry):
        for cc in range(D // 16):
            rows[r, pl.ds(cc * 16, 16)] = jnp.zeros((16,), jnp.float32)
        return carry

    lax.fori_loop(0, SCB, _zero_row, 0)
    base = t * ROWS_PER_TEC
    for j in range(ROWS_PER_TEC // SCB):
        pltpu.sync_copy(rows, aggsh.at[pl.ds(base + j * SCB, SCB)])
    plsc.subcore_barrier()

    def _block(i, carry):
        j = t + i * NSUB

        @pl.when(j < CBLK)
        def _():
            off = j * SCB

            @pl.when(c == 0)
            def _():
                pltpu.sync_copy(rcva_hbm.at[pl.ds(off, SCB)], ridx)
                pltpu.sync_copy(ya_hbm.at[pl.ds(off, SCB)], rows)

            @pl.when(c == 1)
            def _():
                pltpu.sync_copy(rcvb_hbm.at[pl.ds(off, SCB)], ridx)
                pltpu.sync_copy(yb_hbm.at[pl.ds(off, SCB)], rows)

            pltpu.sync_copy(rows, aggsh.at[ridx], add=True)

        return carry

    lax.fori_loop(0, (CBLK + NSUB - 1) // NSUB, _block, 0)
    plsc.subcore_barrier()
    pltpu.sync_copy(aggsh.at[pl.ds(base, ROWS_PER_TEC)],
                    out_hbm.at[c, pl.ds(base, ROWS_PER_TEC)])


def _final_body(agg0_ref, agg1_ref, sc_ref, out_ref):
    tot = (agg0_ref[0, :N] + agg0_ref[1, :N]
           + agg1_ref[0, :N] + agg1_ref[1, :N])
    out_ref[:] = _silu(tot) + sc_ref[:]


def _final(agg0, agg1, sc):
    return pl.pallas_call(
        _final_body,
        out_shape=jax.ShapeDtypeStruct((N, D), jnp.float32),
    )(agg0, agg1, sc)


def kernel(vectors, node_feats, node_specie, senders, receivers,
           W_up, w_sh, W1, W2, W3, W_down, W_skip):
    h, sc = _node_prep(node_feats,
                       node_specie.reshape(N, 1).astype(jnp.int32),
                       W_up, W_skip)
    snd = senders.astype(jnp.int32)
    rcv = receivers.astype(jnp.int32)
    vt = vectors.T
    Q1 = jnp.zeros((9, 128), jnp.float32)
    Q1 = Q1.at[:8, :64].set(W1).at[8, 64].set(1.0)
    w_sh2d = w_sh.reshape(1, 15)

    gather = _make_sc_gather()
    zpad = jnp.zeros((ECHP - ECH,), jnp.int32)
    vpad = jnp.zeros((3, ECHP - ECH), jnp.float32)
    ys = []
    for ci in range(NC):
        lo = ci * ECH
        snd_p = jnp.concatenate([lax.slice(snd, (lo,), (lo + ECH,)), zpad])
        msgs = gather(h, snd_p)
        vt_p = jnp.concatenate(
            [lax.slice(vt, (0, lo), (3, lo + ECH)), vpad], axis=1)
        ys.append(_edge_tc(vt_p, msgs, w_sh2d, Q1, W2, W3, W_down))

    scatter = _make_sc_scatter()
    rchunk = [lax.slice(rcv, (ci * ECH,), ((ci + 1) * ECH,))
              for ci in range(NC)]
    agg0 = scatter(ys[0], ys[1], rchunk[0], rchunk[1])
    agg1 = scatter(ys[2], ys[3], rchunk[2], rchunk[3])
    return _final(agg0, agg1, sc)


# speedup vs baseline: 1.0765x; 1.0765x over previous
import functools
import math

import jax
import jax.numpy as jnp
from jax import lax
from jax.experimental import pallas as pl
from jax.experimental.pallas import tpu as pltpu
from jax.experimental.pallas import tpu_sc as plsc

N = 10000
E = 320000
D = 128
NC = 4
ECH = E // NC
EB = 3200
SCB = 128
CBLK = ECH // SCB
NCORE = 2
NSUB = 16
NWORK = NCORE * NSUB
NPAD = 10240
ROWS_PER_TEC = NPAD // NSUB
INV_SQRT_AVG = 1.0 / math.sqrt(32.0)


def _silu(x):
    return x * jax.nn.sigmoid(x)


def _node_prep_body(nf_ref, sp_ref, wup_ref, wskip_ref, h_ref, sc_ref):
    nf = nf_ref[:]
    h_ref[:] = jnp.dot(nf, wup_ref[:], preferred_element_type=jnp.float32)
    sp = sp_ref[:]
    acc = jnp.zeros_like(nf)
    for k in range(5):
        masked = jnp.where(sp == k, nf, 0.0)
        acc = acc + jnp.dot(masked, wskip_ref[k], preferred_element_type=jnp.float32)
    sc_ref[:] = acc


def _node_prep(node_feats, node_specie2d, W_up, W_skip):
    return pl.pallas_call(
        _node_prep_body,
        out_shape=[
            jax.ShapeDtypeStruct((N, D), jnp.float32),
            jax.ShapeDtypeStruct((N, D), jnp.float32),
        ],
    )(node_feats, node_specie2d, W_up, W_skip)


ECHP = 81920
BPW = ECHP // (NWORK * SCB)
NBUF = 4


@functools.cache
def _make_sc_gather():
    mesh = plsc.VectorSubcoreMesh(core_axis_name="c", subcore_axis_name="s",
                                  num_cores=NCORE, num_subcores=NSUB)
    return functools.partial(
        pl.kernel,
        mesh=mesh,
        out_type=jax.ShapeDtypeStruct((ECHP, D), jnp.float32),
        scratch_types=[
            pltpu.VMEM((BPW * SCB,), jnp.int32),
            [pltpu.VMEM((SCB, D), jnp.float32) for _ in range(NBUF)],
            [pltpu.SemaphoreType.DMA for _ in range(NBUF)],
            [pltpu.SemaphoreType.DMA for _ in range(NBUF)],
        ],
    )(_sc_gather_body)


def _sc_gather_body(h_hbm, snd_hbm, out_hbm, sidx, rows, gsem, osem):
    c = lax.axis_index("c")
    t = lax.axis_index("s")
    w = t * NCORE + c
    base = w * (BPW * SCB)
    pltpu.sync_copy(snd_hbm.at[pl.ds(base, BPW * SCB)], sidx)

    def _gather(k, p):
        return pltpu.async_copy(
            h_hbm.at[sidx.at[pl.ds(k * SCB, SCB)]], rows[p], gsem[p])

    for p in range(NBUF):
        _gather(p, p)

    def _group(g, carry):
        for p in range(NBUF):
            k = g * NBUF + p
            dst = out_hbm.at[pl.ds(base + k * SCB, SCB)]
            pltpu.make_async_copy(dst, rows[p], gsem[p]).wait()
            pltpu.async_copy(rows[p], dst, osem[p]).wait()

            @pl.when(k + NBUF < BPW)
            def _():
                _gather(k + NBUF, p)

        return carry

    lax.fori_loop(0, BPW // NBUF, _group, 0)


_SH_COEF = [
    math.sqrt(3.0), math.sqrt(3.0), math.sqrt(3.0),
    math.sqrt(15.0), math.sqrt(15.0), math.sqrt(5.0) / 2.0,
    math.sqrt(15.0), math.sqrt(15.0) / 2.0,
    math.sqrt(35.0 / 8.0), math.sqrt(105.0), math.sqrt(21.0 / 8.0),
    math.sqrt(7.0) / 2.0, math.sqrt(21.0 / 8.0), math.sqrt(105.0) / 2.0,
    math.sqrt(35.0 / 8.0),
]


def _edge_tc_body(vt_ref, msgs_ref, wsh_ref, q1_ref, w2_ref, w3_ref, wd_ref,
                  y_ref):
    x = vt_ref[0:1]
    y = vt_ref[1:2]
    z = vt_ref[2:3]
    x2 = x * x + y * y + z * z
    l = jnp.sqrt(jnp.where(x2 == 0.0, 1.0, x2))
    invl = 1.0 / l
    ns = (lax.broadcasted_iota(jnp.int32, (8, 1), 0) + 1).astype(jnp.float32)
    l2 = l * l
    l5 = l2 * l2 * l
    env = 1.0 - 21.0 * l5 + 35.0 * l5 * l - 15.0 * l5 * l2
    cutoff = jnp.where(l < 1.0, env, 0.0)
    radT = (jnp.sqrt(2.0) * jnp.sin(ns * (jnp.pi * l))) * (cutoff * invl)
    ux, uy, uz = x * invl, y * invl, z * invl
    comps = [
        ux, uy, uz,
        ux * uy, uy * uz, 3.0 * uz * uz - 1.0, ux * uz, ux * ux - uy * uy,
        uy * (3.0 * ux * ux - uy * uy), ux * uy * uz,
        uy * (5.0 * uz * uz - 1.0), uz * (5.0 * uz * uz - 3.0),
        ux * (5.0 * uz * uz - 1.0), uz * (ux * ux - uy * uy),
        ux * (ux * ux - 3.0 * uy * uy),
    ]
    s = jnp.zeros_like(x)
    for k in range(15):
        s = s + (wsh_ref[0:1, k:k + 1] * _SH_COEF[k]) * comps[k]
    p = jnp.concatenate([radT, s], axis=0)
    r = lax.dot_general(p, q1_ref[:], (((0,), (0,)), ((), ())),
                        preferred_element_type=jnp.float32)
    m = _silu(r[:, :64])
    scol = r[:, 64:65]
    m = _silu(jnp.dot(m, w2_ref[:], preferred_element_type=jnp.float32))
    m = jnp.dot(m, w3_ref[:], preferred_element_type=jnp.float32)
    m = m * INV_SQRT_AVG
    msgs = msgs_ref[:]
    wd = wd_ref[:]
    y_ref[:] = (
        jnp.dot(msgs * m[:, :D], wd[:D], preferred_element_type=jnp.float32)
        + jnp.dot(msgs * (m[:, D:] * scol), wd[D:],
                  preferred_element_type=jnp.float32))


def _edge_tc(vectors_t, msgs, w_sh2d, Q1, W2, W3, W_down):
    grid = (ECH // EB,)
    return pl.pallas_call(
        _edge_tc_body,
        grid=grid,
        in_specs=[
            pl.BlockSpec((3, EB), lambda i: (0, i)),
            pl.BlockSpec((EB, D), lambda i: (i, 0)),
            pl.BlockSpec((1, 15), lambda i: (0, 0)),
            pl.BlockSpec((9, 128), lambda i: (0, 0)),
            pl.BlockSpec((64, 64), lambda i: (0, 0)),
            pl.BlockSpec((64, 2 * D), lambda i: (0, 0)),
            pl.BlockSpec((2 * D, D), lambda i: (0, 0)),
        ],
        out_specs=pl.BlockSpec((EB, D), lambda i: (i, 0)),
        out_shape=jax.ShapeDtypeStruct((ECHP, D), jnp.float32),
    )(vectors_t, msgs, w_sh2d, Q1, W2, W3, W_down)


@functools.cache
def _make_sc_scatter():
    mesh = plsc.VectorSubcoreMesh(core_axis_name="c", subcore_axis_name="s",
                                  num_cores=NCORE, num_subcores=NSUB)
    return functools.partial(
        pl.kernel,
        mesh=mesh,
        out_type=jax.ShapeDtypeStruct((NCORE, NPAD, D), jnp.float32),
        scratch_types=[
            pltpu.VMEM((SCB,), jnp.int32),
            pltpu.VMEM((SCB, D), jnp.float32),
            pltpu.VMEM_SHARED((NPAD, D), jnp.float32),
        ],
    )(_sc_scatter_body)


def _sc_scatter_body(ya_hbm, yb_hbm, rcva_hbm, rcvb_hbm, out_hbm,
                     ridx, rows, aggsh):
    c = lax.axis_index("c")
    t = lax.axis_index("s")

    def _zero_row(r, carry):
        for cc in range(D // 16):
            rows[r, pl.ds(cc * 16, 16)] = jnp.zeros((16,), jnp.float32)
        return carry

    lax.fori_loop(0, SCB, _zero_row, 0)
    base = t * ROWS_PER_TEC
    for j in range(ROWS_PER_TEC // SCB):
        pltpu.sync_copy(rows, aggsh.at[pl.ds(base + j * SCB, SCB)])
    plsc.subcore_barrier()

    def _block(i, carry):
        j = t + i * NSUB

        @pl.when(j < CBLK)
        def _():
            off = j * SCB

            @pl.when(c == 0)
            def _():
                pltpu.sync_copy(rcva_hbm.at[pl.ds(off, SCB)], ridx)
                pltpu.sync_copy(ya_hbm.at[pl.ds(off, SCB)], rows)

            @pl.when(c == 1)
            def _():
                pltpu.sync_copy(rcvb_hbm.at[pl.ds(off, SCB)], ridx)
                pltpu.sync_copy(yb_hbm.at[pl.ds(off, SCB)], rows)

            pltpu.sync_copy(rows, aggsh.at[ridx], add=True)

        return carry

    lax.fori_loop(0, (CBLK + NSUB - 1) // NSUB, _block, 0)
    plsc.subcore_barrier()
    pltpu.sync_copy(aggsh.at[pl.ds(base, ROWS_PER_TEC)],
                    out_hbm.at[c, pl.ds(base, ROWS_PER_TEC)])


def _final_body(agg0_ref, agg1_ref, sc_ref, out_ref):
    tot = (agg0_ref[0, :N] + agg0_ref[1, :N]
           + agg1_ref[0, :N] + agg1_ref[1, :N])
    out_ref[:] = _silu(tot) + sc_ref[:]


def _final(agg0, agg1, sc):
    return pl.pallas_call(
        _final_body,
        out_shape=jax.ShapeDtypeStruct((N, D), jnp.float32),
    )(agg0, agg1, sc)


def kernel(vectors, node_feats, node_specie, senders, receivers,
           W_up, w_sh, W1, W2, W3, W_down, W_skip):
    h, sc = _node_prep(node_feats,
                       node_specie.reshape(N, 1).astype(jnp.int32),
                       W_up, W_skip)
    snd = senders.astype(jnp.int32)
    rcv = receivers.astype(jnp.int32)
    vt = vectors.T
    Q1 = jnp.zeros((9, 128), jnp.float32)
    Q1 = Q1.at[:8, :64].set(W1).at[8, 64].set(1.0)
    w_sh2d = w_sh.reshape(1, 15)

    gather = _make_sc_gather()
    zpad = jnp.zeros((ECHP - ECH,), jnp.int32)
    ys = []
    for ci in range(NC):
        lo = ci * ECH
        snd_p = jnp.concatenate([lax.slice(snd, (lo,), (lo + ECH,)), zpad])
        msgs = gather(h, snd_p)
        ys.append(_edge_tc(lax.slice(vt, (0, lo), (3, lo + ECH)),
                           msgs, w_sh2d, Q1, W2, W3, W_down))

    scatter = _make_sc_scatter()
    rchunk = [lax.slice(rcv, (ci * ECH,), ((ci + 1) * ECH,))
              for ci in range(NC)]
    agg0 = scatter(ys[0], ys[1], rchunk[0], rchunk[1])
    agg1 = scatter(ys[2], ys[3], rchunk[2], rchunk[3])
    return _final(agg0, agg1, sc)


# speedup vs baseline: 2.0064x; 1.8638x over previous
import functools
import math

import jax
import jax.numpy as jnp
from jax import lax
from jax.experimental import pallas as pl
from jax.experimental.pallas import tpu as pltpu
from jax.experimental.pallas import tpu_sc as plsc

N = 10000
E = 320000
D = 128
NC = 4
ECH = E // NC
EB = 3200
SCB = 128
CBLK = ECH // SCB
NCORE = 2
NSUB = 16
NWORK = NCORE * NSUB
NPAD = 10240
ROWS_PER_TEC = NPAD // NSUB
INV_SQRT_AVG = 1.0 / math.sqrt(32.0)


def _silu(x):
    return x * jax.nn.sigmoid(x)


def _node_prep_body(nf_ref, sp_ref, wup_ref, wskip_ref, h_ref, sc_ref):
    nf = nf_ref[:]
    h_ref[:] = jnp.dot(nf, wup_ref[:], preferred_element_type=jnp.float32)
    sp = sp_ref[:]
    acc = jnp.zeros_like(nf)
    for k in range(5):
        masked = jnp.where(sp == k, nf, 0.0)
        acc = acc + jnp.dot(masked, wskip_ref[k], preferred_element_type=jnp.float32)
    sc_ref[:] = acc


def _node_prep(node_feats, node_specie2d, W_up, W_skip):
    return pl.pallas_call(
        _node_prep_body,
        out_shape=[
            jax.ShapeDtypeStruct((N, D), jnp.float32),
            jax.ShapeDtypeStruct((N, D), jnp.float32),
        ],
    )(node_feats, node_specie2d, W_up, W_skip)


@functools.cache
def _make_sc_gather():
    mesh = plsc.VectorSubcoreMesh(core_axis_name="c", subcore_axis_name="s",
                                  num_cores=NCORE, num_subcores=NSUB)
    return functools.partial(
        pl.kernel,
        mesh=mesh,
        out_type=jax.ShapeDtypeStruct((ECH, D), jnp.float32),
        scratch_types=[
            [pltpu.VMEM((SCB,), jnp.int32) for _ in range(2)],
            [pltpu.VMEM((SCB, D), jnp.float32) for _ in range(2)],
            [pltpu.SemaphoreType.DMA for _ in range(2)],
        ],
    )(_sc_gather_body)


def _sc_gather_body(h_hbm, snd_hbm, out_hbm, sidx, rows, gsem):
    c = lax.axis_index("c")
    t = lax.axis_index("s")
    w = t * NCORE + c

    def _start(b, p):
        pltpu.sync_copy(snd_hbm.at[pl.ds(b * SCB, SCB)], sidx[p])
        pltpu.async_copy(h_hbm.at[sidx[p]], rows[p], gsem[p])

    _start(w, 0)

    def _pair(g, carry):
        for p in range(2):
            b = w + (2 * g + p) * NWORK
            nb = b + NWORK

            @pl.when(nb < CBLK)
            def _():
                _start(nb, 1 - p)

            @pl.when(b < CBLK)
            def _():
                pltpu.make_async_copy(
                    h_hbm.at[pl.ds(0, SCB)], rows[p], gsem[p]).wait()
                pltpu.sync_copy(rows[p], out_hbm.at[pl.ds(b * SCB, SCB)])

        return carry

    lax.fori_loop(0, (CBLK + 2 * NWORK - 1) // (2 * NWORK), _pair, 0)


_SH_COEF = [
    math.sqrt(3.0), math.sqrt(3.0), math.sqrt(3.0),
    math.sqrt(15.0), math.sqrt(15.0), math.sqrt(5.0) / 2.0,
    math.sqrt(15.0), math.sqrt(15.0) / 2.0,
    math.sqrt(35.0 / 8.0), math.sqrt(105.0), math.sqrt(21.0 / 8.0),
    math.sqrt(7.0) / 2.0, math.sqrt(21.0 / 8.0), math.sqrt(105.0) / 2.0,
    math.sqrt(35.0 / 8.0),
]


def _edge_tc_body(vt_ref, msgs_ref, wsh_ref, q1_ref, w2_ref, w3_ref, wd_ref,
                  y_ref):
    x = vt_ref[0:1]
    y = vt_ref[1:2]
    z = vt_ref[2:3]
    x2 = x * x + y * y + z * z
    l = jnp.sqrt(jnp.where(x2 == 0.0, 1.0, x2))
    invl = 1.0 / l
    ns = (lax.broadcasted_iota(jnp.int32, (8, 1), 0) + 1).astype(jnp.float32)
    l2 = l * l
    l5 = l2 * l2 * l
    env = 1.0 - 21.0 * l5 + 35.0 * l5 * l - 15.0 * l5 * l2
    cutoff = jnp.where(l < 1.0, env, 0.0)
    radT = (jnp.sqrt(2.0) * jnp.sin(ns * (jnp.pi * l))) * (cutoff * invl)
    ux, uy, uz = x * invl, y * invl, z * invl
    comps = [
        ux, uy, uz,
        ux * uy, uy * uz, 3.0 * uz * uz - 1.0, ux * uz, ux * ux - uy * uy,
        uy * (3.0 * ux * ux - uy * uy), ux * uy * uz,
        uy * (5.0 * uz * uz - 1.0), uz * (5.0 * uz * uz - 3.0),
        ux * (5.0 * uz * uz - 1.0), uz * (ux * ux - uy * uy),
        ux * (ux * ux - 3.0 * uy * uy),
    ]
    s = jnp.zeros_like(x)
    for k in range(15):
        s = s + (wsh_ref[0:1, k:k + 1] * _SH_COEF[k]) * comps[k]
    p = jnp.concatenate([radT, s], axis=0)
    r = lax.dot_general(p, q1_ref[:], (((0,), (0,)), ((), ())),
                        preferred_element_type=jnp.float32)
    m = _silu(r[:, :64])
    scol = r[:, 64:65]
    m = _silu(jnp.dot(m, w2_ref[:], preferred_element_type=jnp.float32))
    m = jnp.dot(m, w3_ref[:], preferred_element_type=jnp.float32)
    m = m * INV_SQRT_AVG
    msgs = msgs_ref[:]
    wd = wd_ref[:]
    y_ref[:] = (
        jnp.dot(msgs * m[:, :D], wd[:D], preferred_element_type=jnp.float32)
        + jnp.dot(msgs * (m[:, D:] * scol), wd[D:],
                  preferred_element_type=jnp.float32))


def _edge_tc(vectors_t, msgs, w_sh2d, Q1, W2, W3, W_down):
    grid = (ECH // EB,)
    return pl.pallas_call(
        _edge_tc_body,
        grid=grid,
        in_specs=[
            pl.BlockSpec((3, EB), lambda i: (0, i)),
            pl.BlockSpec((EB, D), lambda i: (i, 0)),
            pl.BlockSpec((1, 15), lambda i: (0, 0)),
            pl.BlockSpec((9, 128), lambda i: (0, 0)),
            pl.BlockSpec((64, 64), lambda i: (0, 0)),
            pl.BlockSpec((64, 2 * D), lambda i: (0, 0)),
            pl.BlockSpec((2 * D, D), lambda i: (0, 0)),
        ],
        out_specs=pl.BlockSpec((EB, D), lambda i: (i, 0)),
        out_shape=jax.ShapeDtypeStruct((ECH, D), jnp.float32),
    )(vectors_t, msgs, w_sh2d, Q1, W2, W3, W_down)


@functools.cache
def _make_sc_scatter():
    mesh = plsc.VectorSubcoreMesh(core_axis_name="c", subcore_axis_name="s",
                                  num_cores=NCORE, num_subcores=NSUB)
    return functools.partial(
        pl.kernel,
        mesh=mesh,
        out_type=jax.ShapeDtypeStruct((NCORE, NPAD, D), jnp.float32),
        scratch_types=[
            [pltpu.VMEM((SCB,), jnp.int32) for _ in range(2)],
            [pltpu.VMEM((SCB, D), jnp.float32) for _ in range(2)],
            [pltpu.SemaphoreType.DMA for _ in range(2)],
            [pltpu.SemaphoreType.DMA for _ in range(2)],
            pltpu.VMEM_SHARED((NPAD, D), jnp.float32),
        ],
    )(_sc_scatter_body)


def _sc_scatter_body(ya_hbm, yb_hbm, rcva_hbm, rcvb_hbm, out_hbm,
                     ridx, rows, isem, ysem, aggsh):
    c = lax.axis_index("c")
    t = lax.axis_index("s")

    def _zero_row(r, carry):
        for cc in range(D // 16):
            rows[0][r, pl.ds(cc * 16, 16)] = jnp.zeros((16,), jnp.float32)
        return carry

    lax.fori_loop(0, SCB, _zero_row, 0)
    base = t * ROWS_PER_TEC
    for j in range(ROWS_PER_TEC // SCB):
        pltpu.sync_copy(rows[0], aggsh.at[pl.ds(base + j * SCB, SCB)])
    plsc.subcore_barrier()

    def _start(j, p):
        off = j * SCB

        @pl.when(c == 0)
        def _():
            pltpu.async_copy(rcva_hbm.at[pl.ds(off, SCB)], ridx[p], isem[p])
            pltpu.async_copy(ya_hbm.at[pl.ds(off, SCB)], rows[p], ysem[p])

        @pl.when(c == 1)
        def _():
            pltpu.async_copy(rcvb_hbm.at[pl.ds(off, SCB)], ridx[p], isem[p])
            pltpu.async_copy(yb_hbm.at[pl.ds(off, SCB)], rows[p], ysem[p])

    _start(t, 0)

    def _pair(g, carry):
        for p in range(2):
            j = t + (2 * g + p) * NSUB
            nj = j + NSUB

            @pl.when(nj < CBLK)
            def _():
                _start(nj, 1 - p)

            @pl.when(j < CBLK)
            def _():
                pltpu.make_async_copy(
                    rcva_hbm.at[pl.ds(0, SCB)], ridx[p], isem[p]).wait()
                pltpu.make_async_copy(
                    ya_hbm.at[pl.ds(0, SCB)], rows[p], ysem[p]).wait()
                pltpu.sync_copy(rows[p], aggsh.at[ridx[p]], add=True)

        return carry

    lax.fori_loop(0, (CBLK + 2 * NSUB - 1) // (2 * NSUB), _pair, 0)
    plsc.subcore_barrier()
    pltpu.sync_copy(aggsh.at[pl.ds(base, ROWS_PER_TEC)],
                    out_hbm.at[c, pl.ds(base, ROWS_PER_TEC)])


def _final_body(agg0_ref, agg1_ref, sc_ref, out_ref):
    tot = (agg0_ref[0, :N] + agg0_ref[1, :N]
           + agg1_ref[0, :N] + agg1_ref[1, :N])
    out_ref[:] = _silu(tot) + sc_ref[:]


def _final(agg0, agg1, sc):
    return pl.pallas_call(
        _final_body,
        out_shape=jax.ShapeDtypeStruct((N, D), jnp.float32),
    )(agg0, agg1, sc)


def kernel(vectors, node_feats, node_specie, senders, receivers,
           W_up, w_sh, W1, W2, W3, W_down, W_skip):
    h, sc = _node_prep(node_feats,
                       node_specie.reshape(N, 1).astype(jnp.int32),
                       W_up, W_skip)
    snd = senders.astype(jnp.int32)
    rcv = receivers.astype(jnp.int32)
    vt = vectors.T
    Q1 = jnp.zeros((9, 128), jnp.float32)
    Q1 = Q1.at[:8, :64].set(W1).at[8, 64].set(1.0)
    w_sh2d = w_sh.reshape(1, 15)

    gather = _make_sc_gather()
    ys = []
    for ci in range(NC):
        lo = ci * ECH
        msgs = gather(h, lax.slice(snd, (lo,), (lo + ECH,)))
        ys.append(_edge_tc(lax.slice(vt, (0, lo), (3, lo + ECH)),
                           msgs, w_sh2d, Q1, W2, W3, W_down))

    scatter = _make_sc_scatter()
    rchunk = [lax.slice(rcv, (ci * ECH,), ((ci + 1) * ECH,))
              for ci in range(NC)]
    agg0 = scatter(ys[0], ys[1], rchunk[0], rchunk[1])
    agg1 = scatter(ys[2], ys[3], rchunk[2], rchunk[3])
    return _final(agg0, agg1, sc)
